# baseline (speedup 1.0000x reference)
import functools

import jax
import jax.numpy as jnp
from jax import lax
from jax.experimental import pallas as pl
from jax.experimental.pallas import tpu as pltpu
from jax.experimental.pallas import tpu_sc as plsc

_VOCAB = 1000000
_EMB = 32
_HID = 128
_NCLS = 10
_B = 16384
_L = 200
_EPS = 1e-5

_NC = 2
_NS = 16
_NW = _NC * _NS
_ROWS_PER_CORE = _B // _NC
_ROWS_PER_W = _B // _NW

_CHUNK = 112
_OFF_O = 88
_N_CH = 2 * _ROWS_PER_W
_STRIDE = 103
_DUMP = _ROWS_PER_CORE

_NBUF = 8
_G1 = 2
_G2 = 6


def _seg_sum_body(content_hbm, table_hbm, zeros_hbm, out_hbm,
                  idxb, dstb, rows, acc, isem, gsem, ssem):
    cid = lax.axis_index("c")
    sid = lax.axis_index("s")
    wid = cid * _NS + sid

    pltpu.sync_copy(zeros_hbm, acc.at[pl.ds(sid * _ROWS_PER_W, _ROWS_PER_W)])

    lane = lax.iota(jnp.int32, 16)
    vdump = jnp.full((16,), _DUMP, jnp.int32)

    def chunk_of(t):
        return (t * _STRIDE) % _N_CH

    def issue_idx(t, k):
        j = chunk_of(t)
        r = j // 2
        gr = wid * _ROWS_PER_W + r
        off = 0 if k % 2 == 0 else _OFF_O
        pltpu.async_copy(content_hbm.at[gr, pl.ds(off, _CHUNK)],
                         idxb.at[k], isem.at[k])
        lr = sid * _ROWS_PER_W + r
        vlr = jnp.full((16,), lr, jnp.int32)
        if k % 2 == 0:
            for i in range(6):
                dstb[k, pl.ds(i * 16, 16)] = vlr
            dstb[k, pl.ds(96, 16)] = jnp.where(lane < 8, vlr, vdump)
        else:
            dstb[k, pl.ds(0, 16)] = vdump
            for i in range(1, 7):
                dstb[k, pl.ds(i * 16, 16)] = vlr

    def wait_idx(k):
        pltpu.make_async_copy(content_hbm.at[0, pl.ds(0, _CHUNK)],
                              idxb.at[k], isem.at[k]).wait()

    def issue_gather(k):
        pltpu.async_copy(table_hbm.at[idxb.at[k]], rows.at[k], gsem.at[k])

    def wait_gather(k):
        pltpu.make_async_copy(table_hbm.at[idxb.at[k]], rows.at[k],
                              gsem.at[k]).wait()

    def issue_scat(k):
        pltpu.async_copy(rows.at[k], acc.at[dstb.at[k]], ssem.at[k], add=True)

    def wait_scat(k):
        pltpu.make_async_copy(rows.at[k], acc.at[dstb.at[k]],
                              ssem.at[k]).wait()

    for t in range(_NBUF):
        issue_idx(t, t)
        if t >= _G1:
            wait_idx(t - _G1)
            issue_gather(t - _G1)
        if t >= _G2:
            wait_gather(t - _G2)
            issue_scat(t - _G2)

    def steady(i, carry):
        g = i * _NBUF
        for k in range(_NBUF):
            t = g + k
            wait_scat(k)
            issue_idx(t, k)
            k1 = (k - _G1) % _NBUF
            wait_idx(k1)
            issue_gather(k1)
            k2 = (k - _G2) % _NBUF
            wait_gather(k2)
            issue_scat(k2)
        return carry

    lax.fori_loop(1, _N_CH // _NBUF, steady, 0)

    for t in range(_N_CH, _N_CH + _G2):
        k = t % _NBUF
        if t < _N_CH + _G1:
            k1 = (k - _G1) % _NBUF
            wait_idx(k1)
            issue_gather(k1)
        k2 = (k - _G2) % _NBUF
        wait_gather(k2)
        issue_scat(k2)
    for k in range(_NBUF):
        wait_scat(k)

    for i in range(7):
        dstb[0, pl.ds(i * 16, 16)] = vdump
    pltpu.async_copy(rows.at[0], acc.at[dstb.at[0]], ssem.at[0], add=True)
    wait_scat(0)
    plsc.subcore_barrier()

    pltpu.sync_copy(
        acc.at[pl.ds(sid * _ROWS_PER_W, _ROWS_PER_W)],
        out_hbm.at[pl.ds(cid * _ROWS_PER_CORE + sid * _ROWS_PER_W, _ROWS_PER_W)])


_seg_sum = functools.partial(
    pl.kernel,
    out_type=jax.ShapeDtypeStruct((_B, _EMB), jnp.float32),
    mesh=plsc.VectorSubcoreMesh(
        core_axis_name="c", subcore_axis_name="s",
        num_cores=_NC, num_subcores=_NS),
    scratch_types=[
        pltpu.VMEM((_NBUF, _CHUNK), jnp.int32),
        pltpu.VMEM((_NBUF, _CHUNK), jnp.int32),
        pltpu.VMEM((_NBUF, _CHUNK, _EMB), jnp.float32),
        pltpu.VMEM_SHARED((_ROWS_PER_CORE + 8, _EMB), jnp.float32),
        pltpu.SemaphoreType.DMA((_NBUF,)),
        pltpu.SemaphoreType.DMA((_NBUF,)),
        pltpu.SemaphoreType.DMA((_NBUF,)),
    ],
    compiler_params=pltpu.CompilerParams(use_tc_tiling_on_sc=False),
)(_seg_sum_body)


def _mlp_body(x_ref, w1_ref, b1_ref, g1_ref, be1_ref,
              w2_ref, b2_ref, g2_ref, be2_ref, out_ref):
    x = x_ref[...] * (1.0 / _L)
    h = jnp.dot(x, w1_ref[...], preferred_element_type=jnp.float32)
    h = h + b1_ref[...]
    mu = jnp.mean(h, axis=0, keepdims=True)
    hc = h - mu
    var = jnp.mean(hc * hc, axis=0, keepdims=True)
    h = hc * lax.rsqrt(var + _EPS) * g1_ref[...] + be1_ref[...]
    h = jnp.tanh(h)
    o = jnp.dot(h, w2_ref[...], preferred_element_type=jnp.float32)
    o = o + b2_ref[...]
    mu2 = jnp.mean(o, axis=0, keepdims=True)
    oc = o - mu2
    var2 = jnp.mean(oc * oc, axis=0, keepdims=True)
    o = oc * lax.rsqrt(var2 + _EPS) * g2_ref[...] + be2_ref[...]
    out_ref[...] = jax.nn.sigmoid(o)


def _mlp(x, w1t, b1, g1, be1, w2t, b2, g2, be2):
    return pl.pallas_call(
        _mlp_body,
        out_shape=jax.ShapeDtypeStruct((_B, _NCLS), jnp.float32),
    )(x, w1t, b1.reshape(1, _HID), g1.reshape(1, _HID), be1.reshape(1, _HID),
      w2t, b2.reshape(1, _NCLS), g2.reshape(1, _NCLS), be2.reshape(1, _NCLS))


def kernel(content, table, W1, b1, g1, be1, W2, b2, g2, be2):
    zeros = jnp.zeros((_ROWS_PER_W, _EMB), jnp.float32)
    sums = _seg_sum(content, table, zeros)
    return _mlp(sums, W1.T, b1, g1, be1, W2.T, b2, g2, be2)


# speedup vs baseline: 1.0173x; 1.0173x over previous
import functools

import jax
import jax.numpy as jnp
from jax import lax
from jax.experimental import pallas as pl
from jax.experimental.pallas import tpu as pltpu
from jax.experimental.pallas import tpu_sc as plsc

_VOCAB = 1000000
_EMB = 32
_HID = 128
_NCLS = 10
_B = 16384
_L = 200
_EPS = 1e-5

_NC = 2
_NS = 16
_NW = _NC * _NS
_ROWS_PER_CORE = _B // _NC
_ROWS_PER_W = _B // _NW

_CHUNK = 112
_OFF_O = 88
_N_CH = 2 * _ROWS_PER_W
_STRIDE = 103
_DUMP = _ROWS_PER_CORE

_NBUF = 8
_G1 = 2
_G2 = 4


def _seg_sum_body(content_hbm, table_hbm, zeros_hbm, out_hbm,
                  idxb, dstb, rows, acc, isem, gsem, ssem):
    cid = lax.axis_index("c")
    sid = lax.axis_index("s")
    wid = cid * _NS + sid

    pltpu.sync_copy(zeros_hbm, acc.at[pl.ds(sid * _ROWS_PER_W, _ROWS_PER_W)])

    lane = lax.iota(jnp.int32, 16)
    vdump = jnp.full((16,), _DUMP, jnp.int32)

    def chunk_of(t):
        return (t * _STRIDE) % _N_CH

    def issue_idx(t, k):
        j = chunk_of(t)
        r = j // 2
        gr = wid * _ROWS_PER_W + r
        off = gr * _L + (0 if k % 2 == 0 else _OFF_O)
        pltpu.async_copy(content_hbm.at[pl.ds(off, _CHUNK)],
                         idxb.at[k], isem.at[k])
        lr = sid * _ROWS_PER_W + r
        vlr = jnp.full((16,), lr, jnp.int32)
        if k % 2 == 0:
            for i in range(6):
                dstb[k, pl.ds(i * 16, 16)] = vlr
            dstb[k, pl.ds(96, 16)] = jnp.where(lane < 8, vlr, vdump)
        else:
            dstb[k, pl.ds(0, 16)] = vdump
            for i in range(1, 7):
                dstb[k, pl.ds(i * 16, 16)] = vlr

    def wait_idx(k):
        pltpu.make_async_copy(content_hbm.at[pl.ds(0, _CHUNK)],
                              idxb.at[k], isem.at[k]).wait()

    def issue_gather(k):
        pltpu.async_copy(table_hbm.at[idxb.at[k]], rows.at[k], gsem.at[k])

    def wait_gather(k):
        pltpu.make_async_copy(table_hbm.at[idxb.at[k]], rows.at[k],
                              gsem.at[k]).wait()

    def issue_scat(k):
        pltpu.async_copy(rows.at[k], acc.at[dstb.at[k]], ssem.at[k], add=True)

    def wait_scat(k):
        pltpu.make_async_copy(rows.at[k], acc.at[dstb.at[k]],
                              ssem.at[k]).wait()

    for t in range(_NBUF):
        issue_idx(t, t)
        if t >= _G1:
            wait_idx(t - _G1)
            issue_gather(t - _G1)
        if t >= _G2:
            wait_gather(t - _G2)
            issue_scat(t - _G2)

    def steady(i, carry):
        g = i * _NBUF
        for k in range(_NBUF):
            t = g + k
            wait_scat(k)
            issue_idx(t, k)
            k1 = (k - _G1) % _NBUF
            wait_idx(k1)
            issue_gather(k1)
            k2 = (k - _G2) % _NBUF
            wait_gather(k2)
            issue_scat(k2)
        return carry

    lax.fori_loop(1, _N_CH // _NBUF, steady, 0)

    for t in range(_N_CH, _N_CH + _G2):
        k = t % _NBUF
        if t < _N_CH + _G1:
            k1 = (k - _G1) % _NBUF
            wait_idx(k1)
            issue_gather(k1)
        k2 = (k - _G2) % _NBUF
        wait_gather(k2)
        issue_scat(k2)
    for k in range(_NBUF):
        wait_scat(k)

    for i in range(7):
        dstb[0, pl.ds(i * 16, 16)] = vdump
    pltpu.async_copy(rows.at[0], acc.at[dstb.at[0]], ssem.at[0], add=True)
    wait_scat(0)
    plsc.subcore_barrier()

    pltpu.sync_copy(
        acc.at[pl.ds(sid * _ROWS_PER_W, _ROWS_PER_W)],
        out_hbm.at[pl.ds(cid * _ROWS_PER_CORE + sid * _ROWS_PER_W, _ROWS_PER_W)])


_seg_sum = functools.partial(
    pl.kernel,
    out_type=jax.ShapeDtypeStruct((_B, _EMB), jnp.float32),
    mesh=plsc.VectorSubcoreMesh(
        core_axis_name="c", subcore_axis_name="s",
        num_cores=_NC, num_subcores=_NS),
    scratch_types=[
        pltpu.VMEM((_NBUF, _CHUNK), jnp.int32),
        pltpu.VMEM((_NBUF, _CHUNK), jnp.int32),
        pltpu.VMEM((_NBUF, _CHUNK, _EMB), jnp.float32),
        pltpu.VMEM_SHARED((_ROWS_PER_CORE + 8, _EMB), jnp.float32),
        pltpu.SemaphoreType.DMA((_NBUF,)),
        pltpu.SemaphoreType.DMA((_NBUF,)),
        pltpu.SemaphoreType.DMA((_NBUF,)),
    ],
    compiler_params=pltpu.CompilerParams(use_tc_tiling_on_sc=False),
)(_seg_sum_body)


def _mlp_body(x_ref, w1_ref, b1_ref, g1_ref, be1_ref,
              w2_ref, b2_ref, g2_ref, be2_ref, out_ref):
    x = x_ref[...] * (1.0 / _L)
    h = jnp.dot(x, w1_ref[...], preferred_element_type=jnp.float32)
    h = h + b1_ref[...]
    mu = jnp.mean(h, axis=0, keepdims=True)
    hc = h - mu
    var = jnp.mean(hc * hc, axis=0, keepdims=True)
    h = hc * lax.rsqrt(var + _EPS) * g1_ref[...] + be1_ref[...]
    h = jnp.tanh(h)
    o = jnp.dot(h, w2_ref[...], preferred_element_type=jnp.float32)
    o = o + b2_ref[...]
    mu2 = jnp.mean(o, axis=0, keepdims=True)
    oc = o - mu2
    var2 = jnp.mean(oc * oc, axis=0, keepdims=True)
    o = oc * lax.rsqrt(var2 + _EPS) * g2_ref[...] + be2_ref[...]
    out_ref[...] = jax.nn.sigmoid(o)


def _mlp(x, w1t, b1, g1, be1, w2t, b2, g2, be2):
    return pl.pallas_call(
        _mlp_body,
        out_shape=jax.ShapeDtypeStruct((_B, _NCLS), jnp.float32),
    )(x, w1t, b1.reshape(1, _HID), g1.reshape(1, _HID), be1.reshape(1, _HID),
      w2t, b2.reshape(1, _NCLS), g2.reshape(1, _NCLS), be2.reshape(1, _NCLS))


def kernel(content, table, W1, b1, g1, be1, W2, b2, g2, be2):
    zeros = jnp.zeros((_ROWS_PER_W, _EMB), jnp.float32)
    sums = _seg_sum(content.reshape(-1), table, zeros)
    return _mlp(sums, W1.T, b1, g1, be1, W2.T, b2, g2, be2)
